# Initial kernel scaffold; baseline (speedup 1.0000x reference)
import jax
import jax.numpy as jnp
from jax.experimental import pallas as pl


def kernel(tokens, weight):
    raise NotImplementedError("write your pallas kernel here")


# speedup vs baseline: 6.5102x; 6.5102x over previous
import jax
import jax.numpy as jnp
from jax.experimental import pallas as pl
from jax.experimental.pallas import tpu as pltpu
from jax.experimental.pallas import tpu_sc as plsc

_WINDOW = 128


def kernel(tokens, weight):
    B, T = tokens.shape
    V, D = weight.shape
    N = B * T
    idx = tokens.reshape(1, N)
    mesh = plsc.VectorSubcoreMesh(core_axis_name="c", subcore_axis_name="s")

    @pl.kernel(out_type=jax.ShapeDtypeStruct((N, D), weight.dtype), mesh=mesh)
    def gather_kernel(w_hbm, i_hbm, o_hbm):
        def body(i_vmem, o_vmem):
            pltpu.sync_copy(w_hbm.at[i_vmem.at[0]], o_vmem)

        pltpu.emit_pipeline(
            body,
            grid=(N // _WINDOW,),
            in_specs=[pl.BlockSpec((1, _WINDOW), index_map=lambda i: (0, i))],
            out_specs=[pl.BlockSpec((_WINDOW, D), index_map=lambda i: (i, 0))],
            core_axis_name=("c", "s"),
            dimension_semantics=(pltpu.PARALLEL,),
        )(i_hbm, o_hbm)

    out = gather_kernel(weight, idx)
    return out.reshape(B, T, D)


# speedup vs baseline: 7.7165x; 1.1853x over previous
import functools

import jax
import jax.numpy as jnp
from jax.experimental import pallas as pl
from jax.experimental.pallas import tpu as pltpu
from jax.experimental.pallas import tpu_sc as plsc

_WINDOW = 128
_K = 2


def kernel(tokens, weight):
    B, T = tokens.shape
    V, D = weight.shape
    N = B * T
    idx = tokens.reshape(N // _WINDOW, _WINDOW)
    mesh = plsc.VectorSubcoreMesh(core_axis_name="c", subcore_axis_name="s")

    @functools.partial(
        pl.kernel,
        out_type=jax.ShapeDtypeStruct((N, D), weight.dtype),
        mesh=mesh,
        scratch_types=[pltpu.SemaphoreType.DMA],
    )
    def gather_kernel(w_hbm, i_hbm, o_hbm, sem):
        def body(i_vmem, o_vmem):
            copies = [
                pltpu.make_async_copy(
                    w_hbm.at[i_vmem.at[j]],
                    o_vmem.at[pl.ds(j * _WINDOW, _WINDOW)],
                    sem,
                )
                for j in range(_K)
            ]
            for c in copies:
                c.start()
            for c in copies:
                c.wait()

        pltpu.emit_pipeline(
            body,
            grid=(N // (_K * _WINDOW),),
            in_specs=[pl.BlockSpec((_K, _WINDOW), index_map=lambda i: (i, 0))],
            out_specs=[pl.BlockSpec((_K * _WINDOW, D), index_map=lambda i: (i, 0))],
            core_axis_name=("c", "s"),
            dimension_semantics=(pltpu.PARALLEL,),
        )(i_hbm, o_hbm)

    out = gather_kernel(weight, idx)
    return out.reshape(B, T, D)
